# baseline (speedup 1.0000x reference)
import jax
import jax.numpy as jnp
from jax import lax
from jax.experimental import pallas as pl
from jax.experimental.pallas import tpu as pltpu, tpu_sc as plsc

NC = 2
NS = 16
L = 16
NW = NC * NS
B = 16384
F = 16
BPW = B // NW
CHUNK = 128
NCHUNK = BPW // CHUNK
G = 8
NG = BPW // G
RT = 7813
NR = 2 * RT * 8


def _sc_body(u_hbm, i_hbm, pr_hbm, qr_hbm, bu_hbm, bi_hbm, out_hbm,
             uidx_v, iidx_v, pidx_v, qidx_v, pbuf_v, qbuf_v,
             bu_v, bi_v, out_v, sem0, sem1, gsem):
    wid = lax.axis_index("s") * NC + lax.axis_index("c")
    base = wid * BPW
    pltpu.sync_copy(u_hbm.at[pl.ds(base, BPW)], uidx_v.at[pl.ds(0, BPW)])
    pltpu.sync_copy(i_hbm.at[pl.ds(base, BPW)], iidx_v.at[pl.ds(0, BPW)])

    gdescs = []
    for j in range(NCHUNK):
        s = pl.ds(j * CHUNK, CHUNK)
        gdescs.append(pltpu.async_copy(bu_hbm.at[uidx_v.at[s]], bu_v.at[s], gsem))
        gdescs.append(pltpu.async_copy(bi_hbm.at[iidx_v.at[s]], bi_v.at[s], gsem))

    lane = lax.iota(jnp.int32, L)
    K = (lane >> 3) * (RT * 8) + (lane & 7)
    sems = (sem0, sem1)

    def fire(g, par):
        uv = uidx_v[pl.ds(g * G, L)]
        iv = iidx_v[pl.ds(g * G, L)]
        sem = sems[par]
        for j in range(G):
            pidx_v[par, pl.ds(j * L, L)] = K + (uv[j] >> 7) * 8
            qidx_v[par, pl.ds(j * L, L)] = K + (iv[j] >> 7) * 8
        pltpu.async_copy(pr_hbm.at[pidx_v.at[par]], pbuf_v.at[par], sem)
        pltpu.async_copy(qr_hbm.at[qidx_v.at[par]], qbuf_v.at[par], sem)

    def proc(g, par, half, acc):
        uv = uidx_v[pl.ds(g * G, L)]
        iv = iidx_v[pl.ds(g * G, L)]
        sem = sems[par]
        pltpu.make_async_copy(pr_hbm.at[pl.ds(0, G * L)], pbuf_v.at[par],
                              sem).wait()
        pltpu.make_async_copy(qr_hbm.at[pl.ds(0, G * L)], qbuf_v.at[par],
                              sem).wait()
        for j in range(G):
            lu = jnp.full((L,), uv[j] & 127, jnp.int32)
            li = jnp.full((L,), iv[j] & 127, jnp.int32)
            pv = plsc.load_gather(pbuf_v.at[par], [j * L + lane, lu])
            qv = plsc.load_gather(qbuf_v.at[par], [j * L + lane, li])
            s = jnp.sum(pv * qv)
            acc = jnp.where(lane == half + j, acc + s, acc)
        return acc

    for d in gdescs:
        d.wait()

    fire(0, 0)

    def pair(k, carry):
        g0 = 2 * k
        g1 = g0 + 1
        fire(g1, 1)
        blk = pl.ds(k * L, L)
        acc = bu_v[blk] + bi_v[blk]
        acc = proc(g0, 0, 0, acc)

        @pl.when(g0 + 2 < NG)
        def _():
            fire(g0 + 2, 0)

        acc = proc(g1, 1, G, acc)
        out_v[blk] = acc
        return carry

    lax.fori_loop(0, NG // 2, pair, 0)
    pltpu.sync_copy(out_v, out_hbm.at[pl.ds(base, BPW)])


def _tile_rows(T):
    Tp = jnp.pad(T, ((0, RT * 128 - T.shape[0]), (0, 0)))
    return (Tp.T.reshape(2, 8, RT, 128)
            .transpose(0, 2, 1, 3)
            .reshape(NR, 128))


def kernel(user_id, item_id, P, Q, Bu, Bi):
    u = user_id.reshape(-1)
    i = item_id.reshape(-1)
    bu = Bu.reshape(-1)
    bi = Bi.reshape(-1)
    pr = _tile_rows(P)
    qr = _tile_rows(Q)
    mesh = plsc.VectorSubcoreMesh(core_axis_name="c", subcore_axis_name="s",
                                  num_cores=NC, num_subcores=NS)
    out = pl.kernel(
        _sc_body,
        out_type=jax.ShapeDtypeStruct((B,), jnp.float32),
        mesh=mesh,
        compiler_params=pltpu.CompilerParams(needs_layout_passes=False),
        scratch_types=[
            pltpu.VMEM((BPW + L,), jnp.int32),
            pltpu.VMEM((BPW + L,), jnp.int32),
            pltpu.VMEM((2, G * L), jnp.int32),
            pltpu.VMEM((2, G * L), jnp.int32),
            pltpu.VMEM((2, G * L, 128), jnp.float32),
            pltpu.VMEM((2, G * L, 128), jnp.float32),
            pltpu.VMEM((BPW,), jnp.float32),
            pltpu.VMEM((BPW,), jnp.float32),
            pltpu.VMEM((BPW,), jnp.float32),
            pltpu.SemaphoreType.DMA,
            pltpu.SemaphoreType.DMA,
            pltpu.SemaphoreType.DMA,
        ],
    )(u, i, pr, qr, bu, bi)
    return out.reshape(B, 1)
